# baseline (speedup 1.0000x reference)
import functools

import jax
import jax.numpy as jnp
from jax import lax
from jax.experimental import pallas as pl
from jax.experimental.pallas import tpu as pltpu
from jax.experimental.pallas import tpu_sc as plsc

NC, NS = 2, 16
NW = NC * NS
EB = 128

N_SPOT, N_CAT, N_WORD = 10000, 200, 5000
P_SPOT, P_CAT, P_WORD = 10112, 256, 5120
OFF_RH, OFF_RR = 0, P_SPOT
OFF_HAS = 2 * P_SPOT
OFF_REL = OFF_HAS + P_CAT
OFF_NEAR = OFF_REL + P_WORD
R_CNT = OFF_NEAR + P_SPOT


def _pad_edges(ei, n_src, n_dst, mult=EB * NW * 2):
    e = ei.shape[1]
    ep = -(-e // mult) * mult
    ar = jnp.arange(ep - e, dtype=jnp.int32)
    src = jnp.concatenate([ei[0], ar % n_src]).reshape(-1, EB)
    dst = jnp.concatenate([ei[1], n_dst + (ar % 16)]).reshape(-1, EB)
    return src, dst


def _split4(a, nworkers, nsplit=1):
    return a.reshape(nworkers, nsplit, -1, EB)


def _nsplit(kw, n_dst_pad, d=128):
    for ns in (1, 2, 4, 8):
        kwp = kw // ns
        if kw % ns == 0 and kwp % 2 == 0 and \
           n_dst_pad * d + NS * (2 * kwp * EB + 2 * EB * d) < 1_950_000:
            return ns
    raise ValueError(f"no nsplit fits kw={kw} n_dst_pad={n_dst_pad}")


@functools.lru_cache(maxsize=None)
def _segsum_l0(nb, n_src, n_dst_pad):
    kw = nb // NW
    ns = _nsplit(kw, n_dst_pad)
    kwp = kw // ns
    rz = n_dst_pad // NS
    mesh = plsc.VectorSubcoreMesh(core_axis_name="c", subcore_axis_name="s")

    @functools.partial(
        pl.kernel,
        out_type=jax.ShapeDtypeStruct((NC, n_dst_pad, 128), jnp.float32),
        mesh=mesh,
        scratch_types=[
            pltpu.VMEM((kwp, EB), jnp.int32),
            pltpu.VMEM((kwp, EB), jnp.int32),
            pltpu.VMEM((2, EB, 128), jnp.float32),
            pltpu.VMEM_SHARED((n_dst_pad, 128), jnp.float32),
            pltpu.SemaphoreType.DMA,
            pltpu.SemaphoreType.DMA,
            pltpu.SemaphoreType.DMA,
            pltpu.SemaphoreType.DMA,
        ],
    )
    def k(table, src4d, dst4d, zeros, out, sidx, didx, rows, acc, g0, g1, s0,
          s1):
        c = lax.axis_index("c")
        s = lax.axis_index("s")
        w = c * NS + s
        pltpu.sync_copy(zeros.at[pl.ds(s * rz, rz)], acc.at[pl.ds(s * rz, rz)])
        plsc.subcore_barrier()
        gsem, ssem = (g0, g1), (s0, s1)

        for h in range(ns):
            pltpu.sync_copy(src4d.at[w].at[h], sidx)
            pltpu.sync_copy(dst4d.at[w].at[h], didx)
            for b in range(2):
                pltpu.async_copy(table.at[sidx.at[b]], rows.at[b], gsem[b])

            def step(j2, carry):
                for b in range(2):
                    j = 2 * j2 + b
                    pltpu.make_async_copy(
                        table.at[sidx.at[j]], rows.at[b], gsem[b]).wait()
                    pltpu.async_copy(rows.at[b], acc.at[didx.at[j]], ssem[b],
                                     add=True)
                for b in range(2):
                    j = 2 * j2 + b
                    pltpu.make_async_copy(
                        rows.at[b], acc.at[didx.at[j]], ssem[b]).wait()
                    pltpu.async_copy(
                        table.at[sidx.at[jnp.minimum(j + 2, kwp - 1)]],
                        rows.at[b], gsem[b])
                return carry

            lax.fori_loop(0, kwp // 2, step, 0)
            for b in range(2):
                pltpu.make_async_copy(
                    table.at[sidx.at[0]], rows.at[b], gsem[b]).wait()
        plsc.subcore_barrier()
        pltpu.sync_copy(acc.at[pl.ds(s * rz, rz)], out.at[c].at[pl.ds(s * rz, rz)])

    return k, ns


@functools.lru_cache(maxsize=None)
def _segsum_l1(nb, n_src, n_dst_pad):
    kw = nb // NS
    ns = _nsplit(kw, n_dst_pad)
    kwp = kw // ns
    rz = n_dst_pad // NS
    mesh = plsc.VectorSubcoreMesh(core_axis_name="c", subcore_axis_name="s")

    @functools.partial(
        pl.kernel,
        out_type=jax.ShapeDtypeStruct((NC, n_dst_pad, 128), jnp.float32),
        mesh=mesh,
        scratch_types=[
            pltpu.VMEM((kwp, EB), jnp.int32),
            pltpu.VMEM((kwp, EB), jnp.int32),
            pltpu.VMEM((2, EB, 128), jnp.float32),
            pltpu.VMEM_SHARED((n_dst_pad, 128), jnp.float32),
            pltpu.SemaphoreType.DMA,
            pltpu.SemaphoreType.DMA,
            pltpu.SemaphoreType.DMA,
            pltpu.SemaphoreType.DMA,
        ],
    )
    def k(table2, src4d, dst4d, zeros, out, sidx, didx, rows, acc, g0, g1, s0,
          s1):
        c = lax.axis_index("c")
        s = lax.axis_index("s")
        pltpu.sync_copy(zeros.at[pl.ds(s * rz, rz)], acc.at[pl.ds(s * rz, rz)])
        plsc.subcore_barrier()
        gsem, ssem = (g0, g1), (s0, s1)

        for h in range(ns):
            pltpu.sync_copy(src4d.at[s].at[h], sidx)
            pltpu.sync_copy(dst4d.at[s].at[h], didx)
            for b in range(2):
                pltpu.async_copy(table2.at[c].at[sidx.at[b]], rows.at[b],
                                 gsem[b])

            def step(j2, carry):
                for b in range(2):
                    j = 2 * j2 + b
                    pltpu.make_async_copy(
                        table2.at[c].at[sidx.at[j]], rows.at[b],
                        gsem[b]).wait()
                    pltpu.async_copy(rows.at[b], acc.at[didx.at[j]], ssem[b],
                                     add=True)
                for b in range(2):
                    j = 2 * j2 + b
                    pltpu.make_async_copy(
                        rows.at[b], acc.at[didx.at[j]], ssem[b]).wait()
                    pltpu.async_copy(
                        table2.at[c].at[sidx.at[jnp.minimum(j + 2, kwp - 1)]],
                        rows.at[b], gsem[b])
                return carry

            lax.fori_loop(0, kwp // 2, step, 0)
            for b in range(2):
                pltpu.make_async_copy(
                    table2.at[c].at[sidx.at[0]], rows.at[b], gsem[b]).wait()
        plsc.subcore_barrier()
        pltpu.sync_copy(acc.at[pl.ds(s * rz, rz)], out.at[c].at[pl.ds(s * rz, rz)])

    return k, ns


@functools.lru_cache(maxsize=None)
def _count_kernel(nb):
    kw = nb // NW
    rz = R_CNT // NS
    mesh = plsc.VectorSubcoreMesh(core_axis_name="c", subcore_axis_name="s")

    @functools.partial(
        pl.kernel,
        out_type=jax.ShapeDtypeStruct((NC, R_CNT, 16), jnp.float32),
        mesh=mesh,
        compiler_params=pltpu.CompilerParams(use_tc_tiling_on_sc=False),
        scratch_types=[
            pltpu.VMEM((kw, EB), jnp.int32),
            pltpu.VMEM((EB, 16), jnp.float32),
            pltpu.VMEM_SHARED((R_CNT, 16), jnp.float32),
        ],
    )
    def k(dst4d, zeros, ones_h, out, didx, ones_v, acc):
        c = lax.axis_index("c")
        s = lax.axis_index("s")
        w = c * NS + s
        pltpu.sync_copy(zeros.at[pl.ds(s * rz, rz)], acc.at[pl.ds(s * rz, rz)])
        pltpu.sync_copy(ones_h, ones_v)
        pltpu.sync_copy(dst4d.at[w].at[0], didx)
        plsc.subcore_barrier()

        def step(j, carry):
            pltpu.sync_copy(ones_v, acc.at[didx.at[j]], add=True)
            return carry

        lax.fori_loop(0, kw, step, 0)
        plsc.subcore_barrier()
        pltpu.sync_copy(acc.at[pl.ds(s * rz, rz)], out.at[c].at[pl.ds(s * rz, rz)])

    return k


def _prescale(cnt_p, x_spot):
    br = 2048

    def norm_body(cnt_ref, rvec):
        i = pl.program_id(0)
        cnt = cnt_ref[0] + cnt_ref[1]
        rows = i * br + lax.broadcasted_iota(jnp.int32, (br, 16), 0)
        rvec[...] = jnp.where(rows >= OFF_NEAR,
                              lax.rsqrt(cnt + 1.0),
                              1.0 / jnp.maximum(cnt, 1.0))

    rvec = pl.pallas_call(
        norm_body,
        grid=(-(-R_CNT // br),),
        in_specs=[pl.BlockSpec((2, br, 16), lambda i: (0, i, 0))],
        out_specs=pl.BlockSpec((br, 16), lambda i: (i, 0)),
        out_shape=jax.ShapeDtypeStruct((R_CNT, 16), jnp.float32),
    )(cnt_p)

    rcrh = rvec[OFF_RH:OFF_RH + P_SPOT]
    rcrr = rvec[OFF_RR:OFF_RR + P_SPOT]
    rchas = rvec[OFF_HAS:OFF_HAS + P_CAT]
    rcrel = rvec[OFF_REL:OFF_REL + P_WORD]
    rdinv = rvec[OFF_NEAR:OFF_NEAR + P_SPOT]

    bm = 512

    def xs_body(d_ref, x_ref, xs0):
        xs0[...] = x_ref[...] * d_ref[:, :1]

    xs0 = pl.pallas_call(
        xs_body,
        grid=(-(-N_SPOT // bm),),
        in_specs=[pl.BlockSpec((bm, 16), lambda i: (i, 0)),
                  pl.BlockSpec((bm, 128), lambda i: (i, 0))],
        out_specs=pl.BlockSpec((bm, 128), lambda i: (i, 0)),
        out_shape=jax.ShapeDtypeStruct((N_SPOT, 128), jnp.float32),
    )(rdinv, x_spot)
    return xs0, rcrh, rcrr, rchas, rcrel, rdinv


def _dense0_spot(prh, prr, pnr, x_spot, xs0, rcrh, rcrr, rdinv, wstack, bias):
    bm = 512
    grid = (-(-N_SPOT // bm),)

    def body(prh_r, prr_r, pnr_r, x_r, xs0_r, rcrh_r, rcrr_r, rdinv_r, w_r,
             b_r, ox1, oxs1):
        b1 = (prh_r[0] + prh_r[1]) * rcrh_r[:, :1]
        b2 = (prr_r[0] + prr_r[1]) * rcrr_r[:, :1]
        b4 = (pnr_r[0] + pnr_r[1] + xs0_r[...]) * rdinv_r[:, :1]
        lhs = jnp.concatenate([b1, b2, x_r[...], b4], axis=1)
        y = jnp.dot(lhs, w_r[...], preferred_element_type=jnp.float32)
        y = jnp.maximum(y + b_r[...], 0.0)
        ox1[0] = y[:, :128]
        ox1[1] = y[:, 128:]
        oxs1[0] = rdinv_r[:, :1] * y[:, :128]
        oxs1[1] = rdinv_r[:, :1] * y[:, 128:]

    part = lambda: pl.BlockSpec((2, bm, 128), lambda i: (0, i, 0))
    vec = lambda c: pl.BlockSpec((bm, c), lambda i: (i, 0))
    full = lambda a, b: pl.BlockSpec((a, b), lambda i: (0, 0))
    return pl.pallas_call(
        body,
        grid=grid,
        in_specs=[part(), part(), part(), vec(128), vec(128), vec(16),
                  vec(16), vec(16), full(512, 256), full(1, 256)],
        out_specs=[part(), part()],
        out_shape=[
            jax.ShapeDtypeStruct((2, N_SPOT, 128), jnp.float32),
            jax.ShapeDtypeStruct((2, N_SPOT, 128), jnp.float32),
        ],
    )(prh, prr, pnr, x_spot, xs0, rcrh, rcrr, rdinv, wstack, bias)


def _dense0_small(p, x, rc, wstack, bias, n, bm):
    grid = (-(-n // bm),)

    def body(p_r, x_r, rc_r, w_r, b_r, ox1):
        b1 = (p_r[0] + p_r[1]) * rc_r[:, :1]
        lhs = jnp.concatenate([b1, x_r[...]], axis=1)
        y = jnp.dot(lhs, w_r[...], preferred_element_type=jnp.float32)
        y = jnp.maximum(y + b_r[...], 0.0)
        ox1[0] = y[:, :128]
        ox1[1] = y[:, 128:]

    part = lambda: pl.BlockSpec((2, bm, 128), lambda i: (0, i, 0))
    vec = lambda c: pl.BlockSpec((bm, c), lambda i: (i, 0))
    full = lambda a, b: pl.BlockSpec((a, b), lambda i: (0, 0))
    return pl.pallas_call(
        body,
        grid=grid,
        in_specs=[part(), vec(128), vec(16), full(256, 256), full(1, 256)],
        out_specs=[part()],
        out_shape=[jax.ShapeDtypeStruct((2, n, 128), jnp.float32)],
    )(p, x, rc, wstack, bias)[0]


def _dense1(arh, arr, anr, x1, xs1, rcrh, rcrr, rdinv, wstack, bias, wout,
            bout):
    bm = 512
    grid = (-(-N_SPOT // bm),)

    def body(arh_r, arr_r, anr_r, x1_r, xs1_r, rcrh_r, rcrr_r, rdinv_r, w_r,
             b_r, wo_r, bo_r, o):
        b1l = arh_r[0] * rcrh_r[:, :1]
        b1h = arh_r[1] * rcrh_r[:, :1]
        b2l = arr_r[0] * rcrr_r[:, :1]
        b2h = arr_r[1] * rcrr_r[:, :1]
        b4l = (anr_r[0] + xs1_r[0]) * rdinv_r[:, :1]
        b4h = (anr_r[1] + xs1_r[1]) * rdinv_r[:, :1]
        lhs = jnp.concatenate(
            [b1l, b1h, b2l, b2h, x1_r[0], x1_r[1], b4l, b4h], axis=1)
        y = jnp.dot(lhs, w_r[...], preferred_element_type=jnp.float32)
        y = jnp.maximum(y + b_r[...], 0.0)
        o[...] = jnp.dot(y, wo_r[...], preferred_element_type=jnp.float32) \
            + bo_r[...]

    part = lambda: pl.BlockSpec((2, bm, 128), lambda i: (0, i, 0))
    vec = lambda c: pl.BlockSpec((bm, c), lambda i: (i, 0))
    full = lambda a, b: pl.BlockSpec((a, b), lambda i: (0, 0))
    return pl.pallas_call(
        body,
        grid=grid,
        in_specs=[part(), part(), part(), part(), part(), vec(16), vec(16),
                  vec(16), full(1024, 256), full(1, 256), full(256, 128),
                  full(1, 128)],
        out_specs=pl.BlockSpec((bm, 128), lambda i: (i, 0)),
        out_shape=jax.ShapeDtypeStruct((N_SPOT, 128), jnp.float32),
    )(arh, arr, anr, x1, xs1, rcrh, rcrr, rdinv, wstack, bias, wout, bout)


def kernel(x_spot, x_city, x_category, x_word, edge_index_belong,
           edge_index_reblong, edge_index_has, edge_index_rev_has,
           edge_index_revrelate, edge_index_relate, edge_index_near,
           Wl_0_belong, Wr_0_belong, bl_0_belong, Wl_0_reblong, Wr_0_reblong,
           bl_0_reblong, Wl_0_has, Wr_0_has, bl_0_has, Wl_0_rev_has,
           Wr_0_rev_has, bl_0_rev_has, Wl_0_revrelate, Wr_0_revrelate,
           bl_0_revrelate, Wl_0_relate, Wr_0_relate, bl_0_relate, Wg_0, bg_0,
           Wl_1_belong, Wr_1_belong, bl_1_belong, Wl_1_reblong, Wr_1_reblong,
           bl_1_reblong, Wl_1_has, Wr_1_has, bl_1_has, Wl_1_rev_has,
           Wr_1_rev_has, bl_1_rev_has, Wl_1_revrelate, Wr_1_revrelate,
           bl_1_revrelate, Wl_1_relate, Wr_1_relate, bl_1_relate, Wg_1, bg_1,
           W_out, b_out):
    s_rh, d_rh = _pad_edges(edge_index_rev_has, N_CAT, N_SPOT)
    s_rr, d_rr = _pad_edges(edge_index_revrelate, N_WORD, N_SPOT)
    s_has, d_has = _pad_edges(edge_index_has, N_SPOT, N_CAT)
    s_rel, d_rel = _pad_edges(edge_index_relate, N_SPOT, N_WORD)
    s_nr, d_nr = _pad_edges(edge_index_near, N_SPOT, N_SPOT)

    cdst = jnp.concatenate([
        d_rh + OFF_RH, d_rr + OFF_RR, d_has + OFF_HAS, d_rel + OFF_REL,
        d_nr + OFF_NEAR,
    ], axis=0)
    nb_c = cdst.shape[0]
    if nb_c % NW:
        padb = NW - nb_c % NW
        cdst = jnp.concatenate([
            cdst,
            jnp.full((padb, EB), OFF_NEAR + N_SPOT, dtype=jnp.int32),
        ], axis=0)
        nb_c += padb

    zer_spot = jnp.zeros((P_SPOT, 128), jnp.float32)
    zer_cat = jnp.zeros((P_CAT, 128), jnp.float32)
    zer_word = jnp.zeros((P_WORD, 128), jnp.float32)
    zer_cnt = jnp.zeros((R_CNT, 16), jnp.float32)
    ones_h = jnp.ones((EB, 16), jnp.float32)

    cnt_p = _count_kernel(nb_c)(_split4(cdst, NW), zer_cnt, ones_h)
    xs0, rcrh, rcrr, rchas, rcrel, rdinv = _prescale(cnt_p, x_spot)

    def seg0(table, s2, d2, zer, n_src, n_dst_pad):
        k, ns = _segsum_l0(s2.shape[0], n_src, n_dst_pad)
        return k(table, _split4(s2, NW, ns), _split4(d2, NW, ns), zer)

    def seg1(table2, s2, d2, zer, n_src, n_dst_pad):
        k, ns = _segsum_l1(s2.shape[0], n_src, n_dst_pad)
        return k(table2, _split4(s2, NS, ns), _split4(d2, NS, ns), zer)

    prh = seg0(x_category, s_rh, d_rh, zer_spot, N_CAT, P_SPOT)
    prr = seg0(x_word, s_rr, d_rr, zer_spot, N_WORD, P_SPOT)
    phas = seg0(x_spot, s_has, d_has, zer_cat, N_SPOT, P_CAT)
    prel = seg0(x_spot, s_rel, d_rel, zer_word, N_SPOT, P_WORD)
    pnr = seg0(xs0, s_nr, d_nr, zer_spot, N_SPOT, P_SPOT)

    w0_spot = jnp.concatenate(
        [Wl_0_rev_has, Wl_0_revrelate, Wr_0_rev_has + Wr_0_revrelate, Wg_0],
        axis=0)
    b0_spot = (bl_0_rev_has + bl_0_revrelate + bg_0).reshape(1, 256)
    x1s, xs1 = _dense0_spot(prh, prr, pnr, x_spot, xs0, rcrh, rcrr, rdinv,
                            w0_spot, b0_spot)
    x1c = _dense0_small(phas, x_category, rchas,
                        jnp.concatenate([Wl_0_has, Wr_0_has], axis=0),
                        bl_0_has.reshape(1, 256), N_CAT, 256)
    x1w = _dense0_small(prel, x_word, rcrel,
                        jnp.concatenate([Wl_0_relate, Wr_0_relate], axis=0),
                        bl_0_relate.reshape(1, 256), N_WORD, 512)

    arh = seg1(x1c, s_rh, d_rh, zer_spot, N_CAT, P_SPOT)
    arr = seg1(x1w, s_rr, d_rr, zer_spot, N_WORD, P_SPOT)
    anr = seg1(xs1, s_nr, d_nr, zer_spot, N_SPOT, P_SPOT)

    w1_spot = jnp.concatenate(
        [Wl_1_rev_has, Wl_1_revrelate, Wr_1_rev_has + Wr_1_revrelate, Wg_1],
        axis=0)
    b1_spot = (bl_1_rev_has + bl_1_revrelate + bg_1).reshape(1, 256)
    return _dense1(arh, arr, anr, x1s, xs1, rcrh, rcrr, rdinv, w1_spot,
                   b1_spot, W_out, b_out.reshape(1, 128))


# speedup vs baseline: 1.1878x; 1.1878x over previous
import functools

import jax
import jax.numpy as jnp
from jax import lax
from jax.experimental import pallas as pl
from jax.experimental.pallas import tpu as pltpu
from jax.experimental.pallas import tpu_sc as plsc

NC, NS = 2, 16
NW = NC * NS
EB = 128

NBYTES = EB * 128 * 4

N_SPOT, N_CAT, N_WORD = 10000, 200, 5000
P_SPOT, P_CAT, P_WORD = 10112, 256, 5120
OFF_RH, OFF_RR = 0, P_SPOT
OFF_HAS = 2 * P_SPOT
OFF_REL = OFF_HAS + P_CAT
OFF_NEAR = OFF_REL + P_WORD
R_CNT = OFF_NEAR + P_SPOT


def _pad_edges(ei, n_src, n_dst, mult=EB * NW * 2):
    e = ei.shape[1]
    ep = -(-e // mult) * mult
    ar = jnp.arange(ep - e, dtype=jnp.int32)
    src = jnp.concatenate([ei[0], ar % n_src]).reshape(-1, EB)
    dst = jnp.concatenate([ei[1], n_dst + (ar % 16)]).reshape(-1, EB)
    return src, dst


def _split4(a, nworkers, nsplit=1):
    return a.reshape(nworkers, nsplit, -1, EB)


def _nsplit(kw, n_dst_pad, d=128):
    for ns in (1, 2, 4, 8):
        kwp = kw // ns
        if kw % ns == 0 and kwp % 2 == 0 and \
           n_dst_pad * d + NS * (2 * kwp * EB + 2 * EB * d) < 1_950_000:
            return ns
    raise ValueError(f"no nsplit fits kw={kw} n_dst_pad={n_dst_pad}")


@functools.lru_cache(maxsize=None)
def _segsum_l0(nb, n_src, n_dst_pad):
    kw = nb // NW
    ns = _nsplit(kw, n_dst_pad)
    kwp = kw // ns
    rz = n_dst_pad // NS
    mesh = plsc.VectorSubcoreMesh(core_axis_name="c", subcore_axis_name="s")

    @functools.partial(
        pl.kernel,
        out_type=jax.ShapeDtypeStruct((NC, n_dst_pad, 128), jnp.float32),
        mesh=mesh,
        scratch_types=[
            pltpu.VMEM((kwp, EB), jnp.int32),
            pltpu.VMEM((kwp, EB), jnp.int32),
            pltpu.VMEM((2, EB, 128), jnp.float32),
            pltpu.VMEM_SHARED((n_dst_pad, 128), jnp.float32),
            pltpu.SemaphoreType.DMA,
            pltpu.SemaphoreType.DMA,
            pltpu.SemaphoreType.DMA,
            pltpu.SemaphoreType.DMA,
        ],
    )
    def k(table, src4d, dst4d, zeros, out, sidx, didx, rows, acc, g0, g1, s0,
          s1):
        c = lax.axis_index("c")
        s = lax.axis_index("s")
        w = c * NS + s
        pltpu.sync_copy(zeros.at[pl.ds(s * rz, rz)], acc.at[pl.ds(s * rz, rz)])
        plsc.subcore_barrier()
        gsem, ssem = (g0, g1), (s0, s1)

        for h in range(ns):
            pltpu.sync_copy(src4d.at[w].at[h], sidx)
            pltpu.sync_copy(dst4d.at[w].at[h], didx)
            for b in range(2):
                pltpu.async_copy(table.at[sidx.at[b]], rows.at[b], gsem[b])

            def step(j2, carry):
                for b in range(2):
                    j = 2 * j2 + b
                    pltpu.make_async_copy(
                        table.at[sidx.at[j]], rows.at[b], gsem[b]).wait()
                    pltpu.sync_copy(rows.at[b], acc.at[didx.at[j]], add=True)
                    pltpu.async_copy(
                        table.at[sidx.at[jnp.minimum(j + 2, kwp - 1)]],
                        rows.at[b], gsem[b])
                return carry

            lax.fori_loop(0, kwp // 2, step, 0)
            for b in range(2):
                pltpu.make_async_copy(
                    table.at[sidx.at[0]], rows.at[b], gsem[b]).wait()
        plsc.subcore_barrier()
        pltpu.sync_copy(acc.at[pl.ds(s * rz, rz)], out.at[c].at[pl.ds(s * rz, rz)])

    return k, ns


@functools.lru_cache(maxsize=None)
def _segsum_l1(nb, n_src, n_dst_pad):
    kw = nb // NS
    ns = _nsplit(kw, n_dst_pad)
    kwp = kw // ns
    rz = n_dst_pad // NS
    mesh = plsc.VectorSubcoreMesh(core_axis_name="c", subcore_axis_name="s")

    @functools.partial(
        pl.kernel,
        out_type=jax.ShapeDtypeStruct((NC, n_dst_pad, 128), jnp.float32),
        mesh=mesh,
        scratch_types=[
            pltpu.VMEM((kwp, EB), jnp.int32),
            pltpu.VMEM((kwp, EB), jnp.int32),
            pltpu.VMEM((2, EB, 128), jnp.float32),
            pltpu.VMEM_SHARED((n_dst_pad, 128), jnp.float32),
            pltpu.SemaphoreType.DMA,
            pltpu.SemaphoreType.DMA,
            pltpu.SemaphoreType.DMA,
            pltpu.SemaphoreType.DMA,
        ],
    )
    def k(table2, src4d, dst4d, zeros, out, sidx, didx, rows, acc, g0, g1, s0,
          s1):
        c = lax.axis_index("c")
        s = lax.axis_index("s")
        pltpu.sync_copy(zeros.at[pl.ds(s * rz, rz)], acc.at[pl.ds(s * rz, rz)])
        plsc.subcore_barrier()
        gsem, ssem = (g0, g1), (s0, s1)

        for h in range(ns):
            pltpu.sync_copy(src4d.at[s].at[h], sidx)
            pltpu.sync_copy(dst4d.at[s].at[h], didx)
            for b in range(2):
                pltpu.async_copy(table2.at[c].at[sidx.at[b]], rows.at[b],
                                 gsem[b])

            def step(j2, carry):
                for b in range(2):
                    j = 2 * j2 + b
                    pltpu.make_async_copy(
                        table2.at[c].at[sidx.at[j]], rows.at[b],
                        gsem[b]).wait()
                    pltpu.sync_copy(rows.at[b], acc.at[didx.at[j]], add=True)
                    pltpu.async_copy(
                        table2.at[c].at[sidx.at[jnp.minimum(j + 2, kwp - 1)]],
                        rows.at[b], gsem[b])
                return carry

            lax.fori_loop(0, kwp // 2, step, 0)
            for b in range(2):
                pltpu.make_async_copy(
                    table2.at[c].at[sidx.at[0]], rows.at[b], gsem[b]).wait()
        plsc.subcore_barrier()
        pltpu.sync_copy(acc.at[pl.ds(s * rz, rz)], out.at[c].at[pl.ds(s * rz, rz)])

    return k, ns


@functools.lru_cache(maxsize=None)
def _count_kernel(nb):
    kw = nb // NW
    rz = R_CNT // NS
    mesh = plsc.VectorSubcoreMesh(core_axis_name="c", subcore_axis_name="s")

    @functools.partial(
        pl.kernel,
        out_type=jax.ShapeDtypeStruct((NC, R_CNT, 16), jnp.float32),
        mesh=mesh,
        compiler_params=pltpu.CompilerParams(use_tc_tiling_on_sc=False),
        scratch_types=[
            pltpu.VMEM((kw, EB), jnp.int32),
            pltpu.VMEM((EB, 16), jnp.float32),
            pltpu.VMEM_SHARED((R_CNT, 16), jnp.float32),
        ],
    )
    def k(dst4d, zeros, ones_h, out, didx, ones_v, acc):
        c = lax.axis_index("c")
        s = lax.axis_index("s")
        w = c * NS + s
        pltpu.sync_copy(zeros.at[pl.ds(s * rz, rz)], acc.at[pl.ds(s * rz, rz)])
        pltpu.sync_copy(ones_h, ones_v)
        pltpu.sync_copy(dst4d.at[w].at[0], didx)
        plsc.subcore_barrier()

        def step(j, carry):
            pltpu.sync_copy(ones_v, acc.at[didx.at[j]], add=True)
            return carry

        lax.fori_loop(0, kw, step, 0)
        plsc.subcore_barrier()
        pltpu.sync_copy(acc.at[pl.ds(s * rz, rz)], out.at[c].at[pl.ds(s * rz, rz)])

    return k


def _prescale(cnt_p, x_spot):
    br = 2048

    def norm_body(cnt_ref, rvec):
        i = pl.program_id(0)
        cnt = cnt_ref[0] + cnt_ref[1]
        rows = i * br + lax.broadcasted_iota(jnp.int32, (br, 16), 0)
        rvec[...] = jnp.where(rows >= OFF_NEAR,
                              lax.rsqrt(cnt + 1.0),
                              1.0 / jnp.maximum(cnt, 1.0))

    rvec = pl.pallas_call(
        norm_body,
        grid=(-(-R_CNT // br),),
        in_specs=[pl.BlockSpec((2, br, 16), lambda i: (0, i, 0))],
        out_specs=pl.BlockSpec((br, 16), lambda i: (i, 0)),
        out_shape=jax.ShapeDtypeStruct((R_CNT, 16), jnp.float32),
    )(cnt_p)

    rcrh = rvec[OFF_RH:OFF_RH + P_SPOT]
    rcrr = rvec[OFF_RR:OFF_RR + P_SPOT]
    rchas = rvec[OFF_HAS:OFF_HAS + P_CAT]
    rcrel = rvec[OFF_REL:OFF_REL + P_WORD]
    rdinv = rvec[OFF_NEAR:OFF_NEAR + P_SPOT]

    bm = 512

    def xs_body(d_ref, x_ref, xs0):
        xs0[...] = x_ref[...] * d_ref[:, :1]

    xs0 = pl.pallas_call(
        xs_body,
        grid=(-(-N_SPOT // bm),),
        in_specs=[pl.BlockSpec((bm, 16), lambda i: (i, 0)),
                  pl.BlockSpec((bm, 128), lambda i: (i, 0))],
        out_specs=pl.BlockSpec((bm, 128), lambda i: (i, 0)),
        out_shape=jax.ShapeDtypeStruct((N_SPOT, 128), jnp.float32),
    )(rdinv, x_spot)
    return xs0, rcrh, rcrr, rchas, rcrel, rdinv


def _dense0_spot(prh, prr, pnr, x_spot, xs0, rcrh, rcrr, rdinv, wstack, bias):
    bm = 512
    grid = (-(-N_SPOT // bm),)

    def body(prh_r, prr_r, pnr_r, x_r, xs0_r, rcrh_r, rcrr_r, rdinv_r, w_r,
             b_r, ox1, oxs1):
        b1 = (prh_r[0] + prh_r[1]) * rcrh_r[:, :1]
        b2 = (prr_r[0] + prr_r[1]) * rcrr_r[:, :1]
        b4 = (pnr_r[0] + pnr_r[1] + xs0_r[...]) * rdinv_r[:, :1]
        lhs = jnp.concatenate([b1, b2, x_r[...], b4], axis=1)
        y = jnp.dot(lhs, w_r[...], preferred_element_type=jnp.float32)
        y = jnp.maximum(y + b_r[...], 0.0)
        ox1[0] = y[:, :128]
        ox1[1] = y[:, 128:]
        oxs1[0] = rdinv_r[:, :1] * y[:, :128]
        oxs1[1] = rdinv_r[:, :1] * y[:, 128:]

    part = lambda: pl.BlockSpec((2, bm, 128), lambda i: (0, i, 0))
    vec = lambda c: pl.BlockSpec((bm, c), lambda i: (i, 0))
    full = lambda a, b: pl.BlockSpec((a, b), lambda i: (0, 0))
    return pl.pallas_call(
        body,
        grid=grid,
        in_specs=[part(), part(), part(), vec(128), vec(128), vec(16),
                  vec(16), vec(16), full(512, 256), full(1, 256)],
        out_specs=[part(), part()],
        out_shape=[
            jax.ShapeDtypeStruct((2, N_SPOT, 128), jnp.float32),
            jax.ShapeDtypeStruct((2, N_SPOT, 128), jnp.float32),
        ],
    )(prh, prr, pnr, x_spot, xs0, rcrh, rcrr, rdinv, wstack, bias)


def _dense0_small(p, x, rc, wstack, bias, n, bm):
    grid = (-(-n // bm),)

    def body(p_r, x_r, rc_r, w_r, b_r, ox1):
        b1 = (p_r[0] + p_r[1]) * rc_r[:, :1]
        lhs = jnp.concatenate([b1, x_r[...]], axis=1)
        y = jnp.dot(lhs, w_r[...], preferred_element_type=jnp.float32)
        y = jnp.maximum(y + b_r[...], 0.0)
        ox1[0] = y[:, :128]
        ox1[1] = y[:, 128:]

    part = lambda: pl.BlockSpec((2, bm, 128), lambda i: (0, i, 0))
    vec = lambda c: pl.BlockSpec((bm, c), lambda i: (i, 0))
    full = lambda a, b: pl.BlockSpec((a, b), lambda i: (0, 0))
    return pl.pallas_call(
        body,
        grid=grid,
        in_specs=[part(), vec(128), vec(16), full(256, 256), full(1, 256)],
        out_specs=[part()],
        out_shape=[jax.ShapeDtypeStruct((2, n, 128), jnp.float32)],
    )(p, x, rc, wstack, bias)[0]


def _dense1(arh, arr, anr, x1, xs1, rcrh, rcrr, rdinv, wstack, bias, wout,
            bout):
    bm = 512
    grid = (-(-N_SPOT // bm),)

    def body(arh_r, arr_r, anr_r, x1_r, xs1_r, rcrh_r, rcrr_r, rdinv_r, w_r,
             b_r, wo_r, bo_r, o):
        b1l = arh_r[0] * rcrh_r[:, :1]
        b1h = arh_r[1] * rcrh_r[:, :1]
        b2l = arr_r[0] * rcrr_r[:, :1]
        b2h = arr_r[1] * rcrr_r[:, :1]
        b4l = (anr_r[0] + xs1_r[0]) * rdinv_r[:, :1]
        b4h = (anr_r[1] + xs1_r[1]) * rdinv_r[:, :1]
        lhs = jnp.concatenate(
            [b1l, b1h, b2l, b2h, x1_r[0], x1_r[1], b4l, b4h], axis=1)
        y = jnp.dot(lhs, w_r[...], preferred_element_type=jnp.float32)
        y = jnp.maximum(y + b_r[...], 0.0)
        o[...] = jnp.dot(y, wo_r[...], preferred_element_type=jnp.float32) \
            + bo_r[...]

    part = lambda: pl.BlockSpec((2, bm, 128), lambda i: (0, i, 0))
    vec = lambda c: pl.BlockSpec((bm, c), lambda i: (i, 0))
    full = lambda a, b: pl.BlockSpec((a, b), lambda i: (0, 0))
    return pl.pallas_call(
        body,
        grid=grid,
        in_specs=[part(), part(), part(), part(), part(), vec(16), vec(16),
                  vec(16), full(1024, 256), full(1, 256), full(256, 128),
                  full(1, 128)],
        out_specs=pl.BlockSpec((bm, 128), lambda i: (i, 0)),
        out_shape=jax.ShapeDtypeStruct((N_SPOT, 128), jnp.float32),
    )(arh, arr, anr, x1, xs1, rcrh, rcrr, rdinv, wstack, bias, wout, bout)


def kernel(x_spot, x_city, x_category, x_word, edge_index_belong,
           edge_index_reblong, edge_index_has, edge_index_rev_has,
           edge_index_revrelate, edge_index_relate, edge_index_near,
           Wl_0_belong, Wr_0_belong, bl_0_belong, Wl_0_reblong, Wr_0_reblong,
           bl_0_reblong, Wl_0_has, Wr_0_has, bl_0_has, Wl_0_rev_has,
           Wr_0_rev_has, bl_0_rev_has, Wl_0_revrelate, Wr_0_revrelate,
           bl_0_revrelate, Wl_0_relate, Wr_0_relate, bl_0_relate, Wg_0, bg_0,
           Wl_1_belong, Wr_1_belong, bl_1_belong, Wl_1_reblong, Wr_1_reblong,
           bl_1_reblong, Wl_1_has, Wr_1_has, bl_1_has, Wl_1_rev_has,
           Wr_1_rev_has, bl_1_rev_has, Wl_1_revrelate, Wr_1_revrelate,
           bl_1_revrelate, Wl_1_relate, Wr_1_relate, bl_1_relate, Wg_1, bg_1,
           W_out, b_out):
    s_rh, d_rh = _pad_edges(edge_index_rev_has, N_CAT, N_SPOT)
    s_rr, d_rr = _pad_edges(edge_index_revrelate, N_WORD, N_SPOT)
    s_has, d_has = _pad_edges(edge_index_has, N_SPOT, N_CAT)
    s_rel, d_rel = _pad_edges(edge_index_relate, N_SPOT, N_WORD)
    s_nr, d_nr = _pad_edges(edge_index_near, N_SPOT, N_SPOT)

    cdst = jnp.concatenate([
        d_rh + OFF_RH, d_rr + OFF_RR, d_has + OFF_HAS, d_rel + OFF_REL,
        d_nr + OFF_NEAR,
    ], axis=0)
    nb_c = cdst.shape[0]
    if nb_c % NW:
        padb = NW - nb_c % NW
        cdst = jnp.concatenate([
            cdst,
            jnp.full((padb, EB), OFF_NEAR + N_SPOT, dtype=jnp.int32),
        ], axis=0)
        nb_c += padb

    zer_spot = jnp.zeros((P_SPOT, 128), jnp.float32)
    zer_cat = jnp.zeros((P_CAT, 128), jnp.float32)
    zer_word = jnp.zeros((P_WORD, 128), jnp.float32)
    zer_cnt = jnp.zeros((R_CNT, 16), jnp.float32)
    ones_h = jnp.ones((EB, 16), jnp.float32)

    cnt_p = _count_kernel(nb_c)(_split4(cdst, NW), zer_cnt, ones_h)
    xs0, rcrh, rcrr, rchas, rcrel, rdinv = _prescale(cnt_p, x_spot)

    def seg0(table, s2, d2, zer, n_src, n_dst_pad):
        k, ns = _segsum_l0(s2.shape[0], n_src, n_dst_pad)
        return k(table, _split4(s2, NW, ns), _split4(d2, NW, ns), zer)

    def seg1(table2, s2, d2, zer, n_src, n_dst_pad):
        k, ns = _segsum_l1(s2.shape[0], n_src, n_dst_pad)
        return k(table2, _split4(s2, NS, ns), _split4(d2, NS, ns), zer)

    prh = seg0(x_category, s_rh, d_rh, zer_spot, N_CAT, P_SPOT)
    prr = seg0(x_word, s_rr, d_rr, zer_spot, N_WORD, P_SPOT)
    phas = seg0(x_spot, s_has, d_has, zer_cat, N_SPOT, P_CAT)
    prel = seg0(x_spot, s_rel, d_rel, zer_word, N_SPOT, P_WORD)
    pnr = seg0(xs0, s_nr, d_nr, zer_spot, N_SPOT, P_SPOT)

    w0_spot = jnp.concatenate(
        [Wl_0_rev_has, Wl_0_revrelate, Wr_0_rev_has + Wr_0_revrelate, Wg_0],
        axis=0)
    b0_spot = (bl_0_rev_has + bl_0_revrelate + bg_0).reshape(1, 256)
    x1s, xs1 = _dense0_spot(prh, prr, pnr, x_spot, xs0, rcrh, rcrr, rdinv,
                            w0_spot, b0_spot)
    x1c = _dense0_small(phas, x_category, rchas,
                        jnp.concatenate([Wl_0_has, Wr_0_has], axis=0),
                        bl_0_has.reshape(1, 256), N_CAT, 256)
    x1w = _dense0_small(prel, x_word, rcrel,
                        jnp.concatenate([Wl_0_relate, Wr_0_relate], axis=0),
                        bl_0_relate.reshape(1, 256), N_WORD, 512)

    arh = seg1(x1c, s_rh, d_rh, zer_spot, N_CAT, P_SPOT)
    arr = seg1(x1w, s_rr, d_rr, zer_spot, N_WORD, P_SPOT)
    anr = seg1(xs1, s_nr, d_nr, zer_spot, N_SPOT, P_SPOT)

    w1_spot = jnp.concatenate(
        [Wl_1_rev_has, Wl_1_revrelate, Wr_1_rev_has + Wr_1_revrelate, Wg_1],
        axis=0)
    b1_spot = (bl_1_rev_has + bl_1_revrelate + bg_1).reshape(1, 256)
    return _dense1(arh, arr, anr, x1s, xs1, rcrh, rcrr, rdinv, w1_spot,
                   b1_spot, W_out, b_out.reshape(1, 128))
